# baseline (speedup 1.0000x reference)
import jax
import jax.numpy as jnp
from jax import lax
from jax.experimental import pallas as pl


def _ce_roll(v, ii, j):
    bitj = (ii & j) != 0
    partner = jnp.where(bitj, jnp.roll(v, j, 0), jnp.roll(v, -j, 0))
    return jnp.where(bitj, jnp.minimum(v, partner),
                     jnp.maximum(v, partner))


def _ce_slice(v, j):
    n, c = v.shape
    g = n // (2 * j)
    r = v.reshape(g, 2 * j, c)
    lo = r[:, :j, :]
    hi = r[:, j:, :]
    mx = jnp.maximum(lo, hi)
    mn = jnp.minimum(lo, hi)
    return jnp.concatenate([mx, mn], axis=1).reshape(n, c)


def _sort_desc(v):
    n, c = v.shape
    ii = lax.broadcasted_iota(jnp.int32, (n, 1), 0)
    lev = n.bit_length() - 1
    for lk in range(1, lev + 1):
        prev = (ii >> (lk - 1)) & 1 if lk > 1 else jnp.zeros_like(ii)
        cur = (ii >> lk) & 1
        sgn = (1 - 2 * (prev ^ cur)).astype(v.dtype)
        v = v * sgn
        for lj in reversed(range(lk)):
            j = 1 << lj
            if j >= 4:
                v = _ce_slice(v, j)
            else:
                v = _ce_roll(v, ii, j)
    return v


def _stage1(x_ref, o_ref):
    v = x_ref[0].T
    v = _sort_desc(v)
    o_ref[0] = v.T


def _stage2(x_ref, o_ref):
    o_ref[0] = _sort_desc(x_ref[0])


def _impl(x, row_blk=256, col_blk=512, interpret=False):
    B, S, C = x.shape
    row_blk = min(row_blk, S)
    col_blk = min(col_blk, C)
    y = pl.pallas_call(
        _stage1,
        grid=(B, S // row_blk),
        in_specs=[pl.BlockSpec((1, row_blk, C), lambda b, r: (b, r, 0))],
        out_specs=pl.BlockSpec((1, row_blk, C), lambda b, r: (b, r, 0)),
        out_shape=jax.ShapeDtypeStruct((B, S, C), x.dtype),
        interpret=interpret,
    )(x)
    z = pl.pallas_call(
        _stage2,
        grid=(B, C // col_blk),
        in_specs=[pl.BlockSpec((1, S, col_blk), lambda b, c: (b, 0, c))],
        out_specs=pl.BlockSpec((1, S, col_blk), lambda b, c: (b, 0, c)),
        out_shape=jax.ShapeDtypeStruct((B, S, C), x.dtype),
        interpret=interpret,
    )(y)
    return z


def kernel(x):
    return _impl(x)


# speedup vs baseline: 1.0959x; 1.0959x over previous
import jax
import jax.numpy as jnp
from jax import lax
from jax.experimental import pallas as pl


def _ce_roll(v, ii, j, k):
    bitj = (ii & j) != 0
    takes_max = bitj ^ ((ii & k) == 0)
    partner = jnp.where(bitj, jnp.roll(v, j, 0), jnp.roll(v, -j, 0))
    return jnp.where(takes_max, jnp.maximum(v, partner),
                     jnp.minimum(v, partner))


def _ce_slice(v, j, k):
    n, c = v.shape
    g = n // (2 * j)
    r = v.reshape(g, 2 * j, c)
    lo = r[:, :j, :]
    hi = r[:, j:, :]
    mx = jnp.maximum(lo, hi)
    mn = jnp.minimum(lo, hi)
    if k == n:
        nlo, nhi = mx, mn
    else:
        per = k // (2 * j)
        a = lax.broadcasted_iota(jnp.int32, (g, 1, 1), 0)
        desc = (a & per) == 0
        nlo = jnp.where(desc, mx, mn)
        nhi = jnp.where(desc, mn, mx)
    return jnp.concatenate([nlo, nhi], axis=1).reshape(n, c)


def _sort_desc(v):
    n = v.shape[0]
    ii = lax.broadcasted_iota(jnp.int32, v.shape, 0)
    lev = n.bit_length() - 1
    for lk in range(1, lev + 1):
        k = 1 << lk
        for lj in reversed(range(lk)):
            j = 1 << lj
            if j >= 4:
                v = _ce_slice(v, j, k)
            else:
                v = _ce_roll(v, ii, j, k)
    return v


def _stage1(x_ref, o_ref):
    v = x_ref[0].T
    v = _sort_desc(v)
    o_ref[0] = v.T


def _stage2(x_ref, o_ref):
    o_ref[0] = _sort_desc(x_ref[0])


def _impl(x, row_blk=256, col_blk=512, interpret=False):
    B, S, C = x.shape
    row_blk = min(row_blk, S)
    col_blk = min(col_blk, C)
    y = pl.pallas_call(
        _stage1,
        grid=(B, S // row_blk),
        in_specs=[pl.BlockSpec((1, row_blk, C), lambda b, r: (b, r, 0))],
        out_specs=pl.BlockSpec((1, row_blk, C), lambda b, r: (b, r, 0)),
        out_shape=jax.ShapeDtypeStruct((B, S, C), x.dtype),
        interpret=interpret,
    )(x)
    z = pl.pallas_call(
        _stage2,
        grid=(B, C // col_blk),
        in_specs=[pl.BlockSpec((1, S, col_blk), lambda b, c: (b, 0, c))],
        out_specs=pl.BlockSpec((1, S, col_blk), lambda b, c: (b, 0, c)),
        out_shape=jax.ShapeDtypeStruct((B, S, C), x.dtype),
        interpret=interpret,
    )(y)
    return z


def kernel(x):
    return _impl(x)
